# Initial kernel scaffold; baseline (speedup 1.0000x reference)
import jax
import jax.numpy as jnp
from jax.experimental import pallas as pl


def kernel(globals_feat, nodes, edges, node_segment_ids, edge_segment_ids, W1, b1, W2, b2, gamma, beta):
    raise NotImplementedError("write your pallas kernel here")


# speedup vs baseline: 7.3483x; 7.3483x over previous
import functools

import jax
import jax.numpy as jnp
from jax import lax
from jax.experimental import pallas as pl
from jax.experimental.pallas import tpu as pltpu

_B = 16
_D = 128


def _seg_body(ids_ref, x_ref, o_ref):
    i = pl.program_id(0)
    ids = ids_ref[0, 0, :]
    r = ids.shape[0]
    onehot = (lax.broadcasted_iota(jnp.int32, (_B, r), 0) == ids[None, :]).astype(
        jnp.float32
    )
    part = jnp.dot(
        onehot,
        x_ref[...],
        preferred_element_type=jnp.float32,
        precision=lax.Precision.HIGHEST,
    )

    @pl.when(i == 0)
    def _init():
        o_ref[...] = part

    @pl.when(i > 0)
    def _acc():
        o_ref[...] += part


def _segment_sum(x, ids, rows_per_block):
    n = x.shape[0]
    nb = n // rows_per_block
    ids3 = ids.reshape(nb, 1, rows_per_block)
    return pl.pallas_call(
        _seg_body,
        grid=(nb,),
        in_specs=[
            pl.BlockSpec((1, 1, rows_per_block), lambda i: (i, 0, 0)),
            pl.BlockSpec((rows_per_block, _D), lambda i: (i, 0)),
        ],
        out_specs=pl.BlockSpec((_B, _D), lambda i: (0, 0)),
        out_shape=jax.ShapeDtypeStruct((_B, _D), jnp.float32),
    )(ids3, x)


def _mlp_body(g_ref, n_ref, e_ref, w1_ref, b1_ref, w2_ref, b2_ref, gm_ref, bt_ref, o_ref):
    h = (
        jnp.dot(g_ref[...], w1_ref[0:_D, :], preferred_element_type=jnp.float32)
        + jnp.dot(n_ref[...], w1_ref[_D : 2 * _D, :], preferred_element_type=jnp.float32)
        + jnp.dot(e_ref[...], w1_ref[2 * _D : 3 * _D, :], preferred_element_type=jnp.float32)
        + b1_ref[...]
    )
    h = jnp.maximum(h, 0.0)
    y = jnp.dot(h, w2_ref[...], preferred_element_type=jnp.float32) + b2_ref[...]
    out = 1.0 / (1.0 + jnp.exp(-y))
    mean = jnp.mean(out, axis=-1, keepdims=True)
    ctr = out - mean
    var = jnp.mean(ctr * ctr, axis=-1, keepdims=True)
    normed = ctr * lax.rsqrt(var + 1e-3)
    o_ref[...] = normed * gm_ref[...] + bt_ref[...]


def _mlp(globals_feat, node_agg, edge_agg, W1, b1, W2, b2, gamma, beta):
    return pl.pallas_call(
        _mlp_body,
        out_shape=jax.ShapeDtypeStruct((_B, _D), jnp.float32),
    )(
        globals_feat,
        node_agg,
        edge_agg,
        W1,
        b1.reshape(1, -1),
        W2,
        b2.reshape(1, -1),
        gamma.reshape(1, -1),
        beta.reshape(1, -1),
    )


def kernel(globals_feat, nodes, edges, node_segment_ids, edge_segment_ids,
           W1, b1, W2, b2, gamma, beta):
    n_pad = 10240
    nodes_p = jnp.zeros((n_pad, _D), jnp.float32).at[: nodes.shape[0]].set(nodes)
    nids_p = jnp.zeros((n_pad,), jnp.int32).at[: node_segment_ids.shape[0]].set(
        node_segment_ids
    )
    node_agg = _segment_sum(nodes_p, nids_p, 2048)
    edge_agg = _segment_sum(edges, edge_segment_ids, 2560)
    return _mlp(globals_feat, node_agg, edge_agg, W1, b1, W2, b2, gamma, beta)
